# Initial kernel scaffold; baseline (speedup 1.0000x reference)
import jax
import jax.numpy as jnp
from jax.experimental import pallas as pl


def kernel(x, router_W, router_b, W1, b1, W2, b2):
    raise NotImplementedError("write your pallas kernel here")


# speedup vs baseline: 2.5191x; 2.5191x over previous
import functools

import jax
import jax.numpy as jnp
from jax.experimental import pallas as pl
from jax.experimental.pallas import tpu as pltpu

T = 2048
D = 1024
F = 4096
E = 8
FB = 512
NF = F // FB


def _gelu_exact(h):
    return h * 0.5 * (1.0 + jax.lax.erf(h * (2.0 ** -0.5)))


def _router_body(x_ref, rw_ref, rb_ref, e0_ref, e1_ref, w0_ref, w1_ref, aux_ref):
    logits = jnp.dot(x_ref[...], rw_ref[...], preferred_element_type=jnp.float32)
    logits = logits + rb_ref[...]
    eio = jax.lax.broadcasted_iota(jnp.int32, (T, E), 1)
    m0 = jnp.max(logits, axis=1, keepdims=True)
    e0 = jnp.min(jnp.where(logits == m0, eio, E), axis=1, keepdims=True)
    masked = jnp.where(eio == e0, -jnp.inf, logits)
    m1 = jnp.max(masked, axis=1, keepdims=True)
    e1 = jnp.min(jnp.where(masked == m1, eio, E), axis=1, keepdims=True)
    t = jnp.exp(m1 - m0)
    w0 = 1.0 / (1.0 + t)
    w1 = 1.0 - w0
    e0_ref[...] = e0
    e1_ref[...] = e1
    w0_ref[...] = w0
    w1_ref[...] = w1
    c0 = jnp.sum((eio == e0).astype(jnp.float32), axis=0, keepdims=True)
    c1 = jnp.sum((eio == e1).astype(jnp.float32), axis=0, keepdims=True)
    mean = (jnp.sum(c0) + jnp.sum(c1)) / (2.0 * E)
    var = (jnp.sum((c0 - mean) ** 2) + jnp.sum((c1 - mean) ** 2)) / (2.0 * E - 1.0)
    aux_ref[...] = jnp.reshape(var / mean * 0.01, (1, 1))


def _router_call(xs, router_W, router_b):
    return pl.pallas_call(
        _router_body,
        grid=(1,),
        in_specs=[
            pl.BlockSpec((T, D), lambda i: (0, 0)),
            pl.BlockSpec((D, E), lambda i: (0, 0)),
            pl.BlockSpec((1, E), lambda i: (0, 0)),
        ],
        out_specs=[
            pl.BlockSpec((T, 1), lambda i: (0, 0)),
            pl.BlockSpec((T, 1), lambda i: (0, 0)),
            pl.BlockSpec((T, 1), lambda i: (0, 0)),
            pl.BlockSpec((T, 1), lambda i: (0, 0)),
            pl.BlockSpec((1, 1), lambda i: (0, 0)),
        ],
        out_shape=[
            jax.ShapeDtypeStruct((T, 1), jnp.int32),
            jax.ShapeDtypeStruct((T, 1), jnp.int32),
            jax.ShapeDtypeStruct((T, 1), jnp.float32),
            jax.ShapeDtypeStruct((T, 1), jnp.float32),
            jax.ShapeDtypeStruct((1, 1), jnp.float32),
        ],
    )(xs, router_W, router_b.reshape(1, E))


def _moe_dense_body(e0_ref, e1_ref, w0_ref, w1_ref, x_ref, w1w_ref, b1_ref,
                    w2w_ref, b2_ref, out_ref):
    e = pl.program_id(0)
    f = pl.program_id(1)
    h = jnp.dot(x_ref[...], w1w_ref[0], preferred_element_type=jnp.float32)
    h = _gelu_exact(h + b1_ref[0])
    y = jnp.dot(h, w2w_ref[0], preferred_element_type=jnp.float32)
    wcol = (jnp.where(e0_ref[...] == e, w0_ref[...], 0.0)
            + jnp.where(e1_ref[...] == e, w1_ref[...], 0.0))
    bias_gate = jnp.where(f == 0, 1.0, 0.0)
    contrib = (y + bias_gate * b2_ref[0]) * wcol

    @pl.when((e == 0) & (f == 0))
    def _init():
        out_ref[...] = contrib

    @pl.when((e > 0) | (f > 0))
    def _acc():
        out_ref[...] += contrib


def _moe_dense_call(e0, e1, w0, w1, xs, W1, b1, W2, b2):
    return pl.pallas_call(
        _moe_dense_body,
        grid=(E, NF),
        in_specs=[
            pl.BlockSpec((T, 1), lambda e, f: (0, 0)),
            pl.BlockSpec((T, 1), lambda e, f: (0, 0)),
            pl.BlockSpec((T, 1), lambda e, f: (0, 0)),
            pl.BlockSpec((T, 1), lambda e, f: (0, 0)),
            pl.BlockSpec((T, D), lambda e, f: (0, 0)),
            pl.BlockSpec((1, D, FB), lambda e, f: (e, 0, f)),
            pl.BlockSpec((1, 1, FB), lambda e, f: (e, 0, f)),
            pl.BlockSpec((1, FB, D), lambda e, f: (e, f, 0)),
            pl.BlockSpec((1, 1, D), lambda e, f: (e, 0, 0)),
        ],
        out_specs=pl.BlockSpec((T, D), lambda e, f: (0, 0)),
        out_shape=jax.ShapeDtypeStruct((T, D), jnp.float32),
    )(e0, e1, w0, w1, xs, W1, b1.reshape(E, 1, F), W2, b2.reshape(E, 1, D))


def kernel(x, router_W, router_b, W1, b1, W2, b2):
    xs = x.reshape(T, D)
    e0, e1, w0, w1, aux = _router_call(xs, router_W, router_b)
    out = _moe_dense_call(e0, e1, w0, w1, xs, W1, b1, W2, b2)
    return out.reshape(x.shape), aux[0, 0]


# speedup vs baseline: 2.6051x; 1.0341x over previous
import functools

import jax
import jax.numpy as jnp
from jax import lax
from jax.experimental import pallas as pl
from jax.experimental.pallas import tpu as pltpu
from jax.experimental.pallas import tpu_sc as plsc

T = 2048
D = 1024
F = 4096
E = 8
FB = 512
NF = F // FB

BM = 256
NBLK = (T * 2) // BM + E
NSLOT = NBLK * BM

NC = 2
NS = 16
NW = NC * NS
TPW = T // NW
L = 16


def _gelu_exact(h):
    return h * 0.5 * (1.0 + jax.lax.erf(h * (2.0 ** -0.5)))


def _router_body(x_ref, rw_ref, rb_ref, e0_ref, e1_ref, w0_ref, w1_ref, aux_ref):
    logits = jnp.dot(x_ref[...], rw_ref[...], preferred_element_type=jnp.float32)
    logits = logits + rb_ref[...]
    eio = jax.lax.broadcasted_iota(jnp.int32, (T, E), 1)
    m0 = jnp.max(logits, axis=1, keepdims=True)
    e0 = jnp.min(jnp.where(logits == m0, eio, E), axis=1, keepdims=True)
    masked = jnp.where(eio == e0, -jnp.inf, logits)
    m1 = jnp.max(masked, axis=1, keepdims=True)
    e1 = jnp.min(jnp.where(masked == m1, eio, E), axis=1, keepdims=True)
    t = jnp.exp(m1 - m0)
    w0 = 1.0 / (1.0 + t)
    w1 = 1.0 - w0
    e0_ref[...] = e0
    e1_ref[...] = e1
    w0_ref[...] = w0
    w1_ref[...] = w1
    c0 = jnp.sum((eio == e0).astype(jnp.float32), axis=0, keepdims=True)
    c1 = jnp.sum((eio == e1).astype(jnp.float32), axis=0, keepdims=True)
    mean = (jnp.sum(c0) + jnp.sum(c1)) / (2.0 * E)
    var = (jnp.sum((c0 - mean) ** 2) + jnp.sum((c1 - mean) ** 2)) / (2.0 * E - 1.0)
    aux_ref[...] = jnp.reshape(var / mean * 0.01, (1, 1))


def _router_call(xs, router_W, router_b):
    return pl.pallas_call(
        _router_body,
        grid=(1,),
        in_specs=[
            pl.BlockSpec((T, D), lambda i: (0, 0)),
            pl.BlockSpec((D, E), lambda i: (0, 0)),
            pl.BlockSpec((1, E), lambda i: (0, 0)),
        ],
        out_specs=[
            pl.BlockSpec((T, 1), lambda i: (0, 0)),
            pl.BlockSpec((T, 1), lambda i: (0, 0)),
            pl.BlockSpec((T, 1), lambda i: (0, 0)),
            pl.BlockSpec((T, 1), lambda i: (0, 0)),
            pl.BlockSpec((1, 1), lambda i: (0, 0)),
        ],
        out_shape=[
            jax.ShapeDtypeStruct((T, 1), jnp.int32),
            jax.ShapeDtypeStruct((T, 1), jnp.int32),
            jax.ShapeDtypeStruct((T, 1), jnp.float32),
            jax.ShapeDtypeStruct((T, 1), jnp.float32),
            jax.ShapeDtypeStruct((1, 1), jnp.float32),
        ],
    )(xs, router_W, router_b.reshape(1, E))


def _dispatch_body(x_hbm, e0_hbm, e1_hbm,
                   xs_hbm, s0_hbm, s1_hbm, be_hbm,
                   e0_v, e1_v, slots_v, xrows_v, be_v, sem):
    wid = lax.axis_index("s") * NC + lax.axis_index("c")
    base = wid * TPW
    my_first = wid * (TPW // L)
    lanes = jax.lax.broadcasted_iota(jnp.int32, (L,), 0)
    zeros = jnp.zeros((L,), jnp.int32)

    pltpu.sync_copy(e0_hbm, e0_v)
    pltpu.sync_copy(e1_hbm, e1_v)

    def chunk_hist(v):
        hist = jnp.zeros((L,), jnp.int32)
        for e in range(E):
            cnt = jnp.full((L,), jnp.sum((v == e).astype(jnp.int32)))
            hist = hist + jnp.where(lanes == e, cnt, zeros)
        return hist

    def count_step(c, carry):
        run, mybase = carry
        snap = jnp.full((L,), c) == jnp.full((L,), my_first)
        mybase = jnp.where(snap, run, mybase)
        h = chunk_hist(e0_v[pl.ds(c * L, L)]) + chunk_hist(e1_v[pl.ds(c * L, L)])
        return run + h, mybase

    tot, mybase = lax.fori_loop(0, T // L, count_step, (zeros, zeros))
    capb = (tot + (BM - 1)) // BM
    endb = lax.cumsum(capb)
    offb = endb - capb
    start = offb * BM + mybase

    @pl.when(wid == 0)
    def _():
        for half in range(NBLK // L + 1):
            bi = lanes + half * L
            acc = jnp.zeros((L,), jnp.int32)
            for e in range(E):
                endb_e = jnp.full((L,), jnp.sum(jnp.where(lanes == e, endb, zeros)))
                acc = acc + (bi >= endb_e).astype(jnp.int32)
            be_v[pl.ds(half * L, L)] = jnp.minimum(acc, jnp.full((L,), E - 1))
        pltpu.sync_copy(be_v, be_hbm)

    run2 = start
    for cc in range(TPW // L):
        c = my_first + cc
        for row, src_v in ((0, e0_v), (1, e1_v)):
            v = src_v[pl.ds(c * L, L)]
            r = jnp.zeros((L,), jnp.int32)
            sb = jnp.zeros((L,), jnp.int32)
            hist = jnp.zeros((L,), jnp.int32)
            for e in range(E):
                m = v == e
                cs = lax.cumsum(m.astype(jnp.int32))
                r = jnp.where(m, cs - 1, r)
                run_e = jnp.full((L,), jnp.sum(jnp.where(lanes == e, run2, zeros)))
                sb = jnp.where(m, run_e, sb)
                cnt = jnp.full((L,), jnp.sum(m.astype(jnp.int32)))
                hist = hist + jnp.where(lanes == e, cnt, zeros)
            slots_v[row, pl.ds(cc * L, L)] = sb + r
            run2 = run2 + hist

    pltpu.sync_copy(slots_v.at[0], s0_hbm.at[pl.ds(base, TPW)])
    pltpu.sync_copy(slots_v.at[1], s1_hbm.at[pl.ds(base, TPW)])

    pltpu.sync_copy(x_hbm.at[pl.ds(base, TPW)], xrows_v)
    pltpu.async_copy(xrows_v, xs_hbm.at[slots_v.at[0]], sem).wait()
    pltpu.async_copy(xrows_v, xs_hbm.at[slots_v.at[1]], sem).wait()


def _dispatch_call(xs, e0, e1):
    mesh = plsc.VectorSubcoreMesh(core_axis_name="c", subcore_axis_name="s")
    fn = functools.partial(
        pl.kernel,
        out_type=[
            jax.ShapeDtypeStruct((NSLOT, D), jnp.float32),
            jax.ShapeDtypeStruct((T,), jnp.int32),
            jax.ShapeDtypeStruct((T,), jnp.int32),
            jax.ShapeDtypeStruct((2 * L,), jnp.int32),
        ],
        mesh=mesh,
        scratch_types=[
            pltpu.VMEM((T,), jnp.int32),
            pltpu.VMEM((T,), jnp.int32),
            pltpu.VMEM((2, TPW), jnp.int32),
            pltpu.VMEM((TPW, D), jnp.float32),
            pltpu.VMEM((2 * L,), jnp.int32),
            pltpu.SemaphoreType.DMA,
        ],
        compiler_params=pltpu.CompilerParams(needs_layout_passes=False),
    )(_dispatch_body)
    return fn(xs, e0, e1)


def _gffn_body(be_ref, xs_ref, w1_ref, b1_ref, w2_ref, b2_ref, out_ref, yacc_ref):
    f = pl.program_id(0)
    blk = pl.program_id(1)
    h = jnp.dot(xs_ref[...], w1_ref[0], preferred_element_type=jnp.float32)
    h = _gelu_exact(h + b1_ref[0])
    y = jnp.dot(h, w2_ref[0], preferred_element_type=jnp.float32)
    row = blk * BM

    @pl.when(f == 0)
    def _():
        yacc_ref[pl.ds(row, BM), :] = y

    @pl.when(f > 0)
    def _():
        yacc_ref[pl.ds(row, BM), :] += y

    @pl.when(f == NF - 1)
    def _():
        out_ref[...] = yacc_ref[pl.ds(row, BM), :] + b2_ref[0]


def _gffn_call(be, x_sorted, W1, b1, W2, b2):
    grid_spec = pltpu.PrefetchScalarGridSpec(
        num_scalar_prefetch=1,
        grid=(NF, NBLK),
        in_specs=[
            pl.BlockSpec((BM, D), lambda f, blk, be: (blk, 0)),
            pl.BlockSpec((1, D, FB), lambda f, blk, be: (be[blk], 0, f)),
            pl.BlockSpec((1, 1, FB), lambda f, blk, be: (be[blk], 0, f)),
            pl.BlockSpec((1, FB, D), lambda f, blk, be: (be[blk], f, 0)),
            pl.BlockSpec((1, 1, D), lambda f, blk, be: (be[blk], 0, 0)),
        ],
        out_specs=pl.BlockSpec(
            (BM, D), lambda f, blk, be: (jnp.where(f == NF - 1, blk, 0), 0)),
        scratch_shapes=[pltpu.VMEM((NSLOT, D), jnp.float32)],
    )
    return pl.pallas_call(
        _gffn_body,
        grid_spec=grid_spec,
        out_shape=jax.ShapeDtypeStruct((NSLOT, D), jnp.float32),
        compiler_params=pltpu.CompilerParams(vmem_limit_bytes=64 * 1024 * 1024),
    )(be, x_sorted, W1, b1.reshape(E, 1, F), W2, b2.reshape(E, 1, D))


CH = 16


def _combine_body(y_hbm, s0_hbm, s1_hbm, w0_hbm, w1_hbm, out_hbm,
                  sl_v, w0_v, w1_v, ya_v, yb_v, sem):
    wid = lax.axis_index("s") * NC + lax.axis_index("c")
    base = wid * TPW
    lanes = jax.lax.broadcasted_iota(jnp.int32, (L,), 0)

    pltpu.sync_copy(s0_hbm.at[pl.ds(base, TPW)], sl_v.at[0])
    pltpu.sync_copy(s1_hbm.at[pl.ds(base, TPW)], sl_v.at[1])
    pltpu.sync_copy(w0_hbm.at[pl.ds(base, TPW)], w0_v)
    pltpu.sync_copy(w1_hbm.at[pl.ds(base, TPW)], w1_v)

    for g in range(TPW // CH):
        ca = pltpu.async_copy(y_hbm.at[sl_v.at[0, pl.ds(g * CH, CH)]], ya_v, sem)
        cb = pltpu.async_copy(y_hbm.at[sl_v.at[1, pl.ds(g * CH, CH)]], yb_v, sem)
        ca.wait()
        cb.wait()
        wa = w0_v[pl.ds(g * CH, CH)]
        wb = w1_v[pl.ds(g * CH, CH)]

        def tok_body(i, _, wa=wa, wb=wb):
            iv = jnp.full((L,), i)
            fz = jnp.zeros((L,), jnp.float32)
            was = jnp.full((L,), jnp.sum(jnp.where(lanes == iv, wa, fz)))
            wbs = jnp.full((L,), jnp.sum(jnp.where(lanes == iv, wb, fz)))
            for j in range(D // L):
                sla = ya_v[i, pl.ds(j * L, L)]
                slb = yb_v[i, pl.ds(j * L, L)]
                ya_v[i, pl.ds(j * L, L)] = was * sla + wbs * slb
            return 0

        lax.fori_loop(0, CH, tok_body, 0)
        pltpu.sync_copy(ya_v, out_hbm.at[pl.ds(base + g * CH, CH)])


def _combine_call(y_sorted, s0, s1, w0, w1):
    mesh = plsc.VectorSubcoreMesh(core_axis_name="c", subcore_axis_name="s")
    fn = functools.partial(
        pl.kernel,
        out_type=jax.ShapeDtypeStruct((T, D), jnp.float32),
        mesh=mesh,
        scratch_types=[
            pltpu.VMEM((2, TPW), jnp.int32),
            pltpu.VMEM((TPW,), jnp.float32),
            pltpu.VMEM((TPW,), jnp.float32),
            pltpu.VMEM((CH, D), jnp.float32),
            pltpu.VMEM((CH, D), jnp.float32),
            pltpu.SemaphoreType.DMA,
        ],
        compiler_params=pltpu.CompilerParams(needs_layout_passes=False),
    )(_combine_body)
    return fn(y_sorted, s0, s1, w0, w1)


def kernel(x, router_W, router_b, W1, b1, W2, b2):
    xs = x.reshape(T, D)
    e0, e1, w0, w1, aux = _router_call(xs, router_W, router_b)
    x_sorted, s0, s1, be = _dispatch_call(xs, e0.reshape(T), e1.reshape(T))
    y_sorted = _gffn_call(be, x_sorted, W1, b1, W2, b2)
    out = _combine_call(y_sorted, s0, s1, w0.reshape(T), w1.reshape(T))
    return out.reshape(x.shape), aux[0, 0]


# speedup vs baseline: 3.3176x; 1.2735x over previous
import functools

import jax
import jax.numpy as jnp
from jax import lax
from jax.experimental import pallas as pl
from jax.experimental.pallas import tpu as pltpu
from jax.experimental.pallas import tpu_sc as plsc

T = 2048
D = 1024
F = 4096
E = 8
FB = 512
NF = F // FB

BM = 256
NBLK = (T * 2) // BM + E
NSLOT = NBLK * BM

NC = 2
NS = 16
NW = NC * NS
TPW = T // NW
L = 16


def _gelu_exact(h):
    return h * 0.5 * (1.0 + jax.lax.erf(h * (2.0 ** -0.5)))


def _router_body(x_ref, rw_ref, rb_ref, e0_ref, e1_ref, w0_ref, w1_ref, aux_ref):
    logits = jnp.dot(x_ref[...], rw_ref[...], preferred_element_type=jnp.float32)
    logits = logits + rb_ref[...]
    eio = jax.lax.broadcasted_iota(jnp.int32, (T, E), 1)
    m0 = jnp.max(logits, axis=1, keepdims=True)
    e0 = jnp.min(jnp.where(logits == m0, eio, E), axis=1, keepdims=True)
    masked = jnp.where(eio == e0, -jnp.inf, logits)
    m1 = jnp.max(masked, axis=1, keepdims=True)
    e1 = jnp.min(jnp.where(masked == m1, eio, E), axis=1, keepdims=True)
    t = jnp.exp(m1 - m0)
    w0 = 1.0 / (1.0 + t)
    w1 = 1.0 - w0
    e0_ref[...] = e0
    e1_ref[...] = e1
    w0_ref[...] = w0
    w1_ref[...] = w1
    c0 = jnp.sum((eio == e0).astype(jnp.float32), axis=0, keepdims=True)
    c1 = jnp.sum((eio == e1).astype(jnp.float32), axis=0, keepdims=True)
    mean = (jnp.sum(c0) + jnp.sum(c1)) / (2.0 * E)
    var = (jnp.sum((c0 - mean) ** 2) + jnp.sum((c1 - mean) ** 2)) / (2.0 * E - 1.0)
    aux_ref[...] = jnp.reshape(var / mean * 0.01, (1, 1))


def _router_call(xs, router_W, router_b):
    return pl.pallas_call(
        _router_body,
        grid=(1,),
        in_specs=[
            pl.BlockSpec((T, D), lambda i: (0, 0)),
            pl.BlockSpec((D, E), lambda i: (0, 0)),
            pl.BlockSpec((1, E), lambda i: (0, 0)),
        ],
        out_specs=[
            pl.BlockSpec((T, 1), lambda i: (0, 0)),
            pl.BlockSpec((T, 1), lambda i: (0, 0)),
            pl.BlockSpec((T, 1), lambda i: (0, 0)),
            pl.BlockSpec((T, 1), lambda i: (0, 0)),
            pl.BlockSpec((1, 1), lambda i: (0, 0)),
        ],
        out_shape=[
            jax.ShapeDtypeStruct((T, 1), jnp.int32),
            jax.ShapeDtypeStruct((T, 1), jnp.int32),
            jax.ShapeDtypeStruct((T, 1), jnp.float32),
            jax.ShapeDtypeStruct((T, 1), jnp.float32),
            jax.ShapeDtypeStruct((1, 1), jnp.float32),
        ],
    )(xs, router_W, router_b.reshape(1, E))


def _dispatch_body(x_hbm, e0_hbm, e1_hbm,
                   xs_hbm, s0_hbm, s1_hbm, be_hbm,
                   e0_v, e1_v, slots_v, xrows_v, be_v, sem):
    wid = lax.axis_index("s") * NC + lax.axis_index("c")
    base = wid * TPW
    my_first = wid * (TPW // L)
    lanes = jax.lax.broadcasted_iota(jnp.int32, (L,), 0)
    zeros = jnp.zeros((L,), jnp.int32)

    pltpu.sync_copy(e0_hbm, e0_v)
    pltpu.sync_copy(e1_hbm, e1_v)

    def chunk_hist(v):
        hist = jnp.zeros((L,), jnp.int32)
        for e in range(E):
            cnt = jnp.full((L,), jnp.sum((v == e).astype(jnp.int32)))
            hist = hist + jnp.where(lanes == e, cnt, zeros)
        return hist

    def count_step(c, carry):
        run, mybase = carry
        snap = jnp.full((L,), c) == jnp.full((L,), my_first)
        mybase = jnp.where(snap, run, mybase)
        h = chunk_hist(e0_v[pl.ds(c * L, L)]) + chunk_hist(e1_v[pl.ds(c * L, L)])
        return run + h, mybase

    tot, mybase = lax.fori_loop(0, T // L, count_step, (zeros, zeros))
    capb = (tot + (BM - 1)) // BM
    endb = lax.cumsum(capb)
    offb = endb - capb
    start = offb * BM + mybase

    @pl.when(wid == 0)
    def _():
        for half in range(NBLK // L + 1):
            bi = lanes + half * L
            acc = jnp.zeros((L,), jnp.int32)
            for e in range(E):
                endb_e = jnp.full((L,), jnp.sum(jnp.where(lanes == e, endb, zeros)))
                acc = acc + (bi >= endb_e).astype(jnp.int32)
            be_v[pl.ds(half * L, L)] = jnp.minimum(acc, jnp.full((L,), E - 1))
        pltpu.sync_copy(be_v, be_hbm)

    run2 = start
    for cc in range(TPW // L):
        c = my_first + cc
        for row, src_v in ((0, e0_v), (1, e1_v)):
            v = src_v[pl.ds(c * L, L)]
            r = jnp.zeros((L,), jnp.int32)
            sb = jnp.zeros((L,), jnp.int32)
            hist = jnp.zeros((L,), jnp.int32)
            for e in range(E):
                m = v == e
                cs = lax.cumsum(m.astype(jnp.int32))
                r = jnp.where(m, cs - 1, r)
                run_e = jnp.full((L,), jnp.sum(jnp.where(lanes == e, run2, zeros)))
                sb = jnp.where(m, run_e, sb)
                cnt = jnp.full((L,), jnp.sum(m.astype(jnp.int32)))
                hist = hist + jnp.where(lanes == e, cnt, zeros)
            slots_v[row, pl.ds(cc * L, L)] = sb + r
            run2 = run2 + hist

    pltpu.sync_copy(slots_v.at[0], s0_hbm.at[pl.ds(base, TPW)])
    pltpu.sync_copy(slots_v.at[1], s1_hbm.at[pl.ds(base, TPW)])

    pltpu.sync_copy(x_hbm.at[pl.ds(base, TPW)], xrows_v)
    pltpu.async_copy(xrows_v, xs_hbm.at[slots_v.at[0]], sem).wait()
    pltpu.async_copy(xrows_v, xs_hbm.at[slots_v.at[1]], sem).wait()


def _dispatch_call(xs, e0, e1):
    mesh = plsc.VectorSubcoreMesh(core_axis_name="c", subcore_axis_name="s")
    fn = functools.partial(
        pl.kernel,
        out_type=[
            jax.ShapeDtypeStruct((NSLOT, D), jnp.float32),
            jax.ShapeDtypeStruct((T,), jnp.int32),
            jax.ShapeDtypeStruct((T,), jnp.int32),
            jax.ShapeDtypeStruct((2 * L,), jnp.int32),
        ],
        mesh=mesh,
        scratch_types=[
            pltpu.VMEM((T,), jnp.int32),
            pltpu.VMEM((T,), jnp.int32),
            pltpu.VMEM((2, TPW), jnp.int32),
            pltpu.VMEM((TPW, D), jnp.float32),
            pltpu.VMEM((2 * L,), jnp.int32),
            pltpu.SemaphoreType.DMA,
        ],
        compiler_params=pltpu.CompilerParams(needs_layout_passes=False),
    )(_dispatch_body)
    return fn(xs, e0, e1)


FH = F // 2


def _gffn_body(be_ref, xs_ref, w1_ref, b1_ref, w2_ref, b2_ref, out_ref):
    j = pl.program_id(1)
    h = jnp.dot(xs_ref[...], w1_ref[0], preferred_element_type=jnp.float32)
    h = _gelu_exact(h + b1_ref[0])
    y = jnp.dot(h, w2_ref[0], preferred_element_type=jnp.float32)

    @pl.when(j == 0)
    def _():
        out_ref[...] = y + b2_ref[0]

    @pl.when(j == 1)
    def _():
        out_ref[...] += y


def _gffn_call(be, x_sorted, W1, b1, W2, b2):
    def _jser(blk, j):
        return jax.lax.bitwise_xor(jax.lax.rem(blk, 2), j)

    grid_spec = pltpu.PrefetchScalarGridSpec(
        num_scalar_prefetch=1,
        grid=(NBLK, 2),
        in_specs=[
            pl.BlockSpec((BM, D), lambda blk, j, be: (blk, 0)),
            pl.BlockSpec((1, D, FH), lambda blk, j, be: (be[blk], 0, _jser(blk, j))),
            pl.BlockSpec((1, 1, FH), lambda blk, j, be: (be[blk], 0, _jser(blk, j))),
            pl.BlockSpec((1, FH, D), lambda blk, j, be: (be[blk], _jser(blk, j), 0)),
            pl.BlockSpec((1, 1, D), lambda blk, j, be: (be[blk], 0, 0)),
        ],
        out_specs=pl.BlockSpec((BM, D), lambda blk, j, be: (blk, 0)),
    )
    return pl.pallas_call(
        _gffn_body,
        grid_spec=grid_spec,
        out_shape=jax.ShapeDtypeStruct((NSLOT, D), jnp.float32),
        compiler_params=pltpu.CompilerParams(vmem_limit_bytes=60 * 1024 * 1024),
    )(be, x_sorted, W1, b1.reshape(E, 1, F), W2, b2.reshape(E, 1, D))


CH = 16


def _combine_body(y_hbm, s0_hbm, s1_hbm, w0_hbm, w1_hbm, out_hbm,
                  sl_v, w0_v, w1_v, ya_v, yb_v, sem):
    wid = lax.axis_index("s") * NC + lax.axis_index("c")
    base = wid * TPW
    lanes = jax.lax.broadcasted_iota(jnp.int32, (L,), 0)

    pltpu.sync_copy(s0_hbm.at[pl.ds(base, TPW)], sl_v.at[0])
    pltpu.sync_copy(s1_hbm.at[pl.ds(base, TPW)], sl_v.at[1])
    pltpu.sync_copy(w0_hbm.at[pl.ds(base, TPW)], w0_v)
    pltpu.sync_copy(w1_hbm.at[pl.ds(base, TPW)], w1_v)

    for g in range(TPW // CH):
        ca = pltpu.async_copy(y_hbm.at[sl_v.at[0, pl.ds(g * CH, CH)]], ya_v, sem)
        cb = pltpu.async_copy(y_hbm.at[sl_v.at[1, pl.ds(g * CH, CH)]], yb_v, sem)
        ca.wait()
        cb.wait()
        wa = w0_v[pl.ds(g * CH, CH)]
        wb = w1_v[pl.ds(g * CH, CH)]

        def tok_body(i, _, wa=wa, wb=wb):
            iv = jnp.full((L,), i)
            fz = jnp.zeros((L,), jnp.float32)
            was = jnp.full((L,), jnp.sum(jnp.where(lanes == iv, wa, fz)))
            wbs = jnp.full((L,), jnp.sum(jnp.where(lanes == iv, wb, fz)))
            for j in range(D // L):
                sla = ya_v[i, pl.ds(j * L, L)]
                slb = yb_v[i, pl.ds(j * L, L)]
                ya_v[i, pl.ds(j * L, L)] = was * sla + wbs * slb
            return 0

        lax.fori_loop(0, CH, tok_body, 0)
        pltpu.sync_copy(ya_v, out_hbm.at[pl.ds(base + g * CH, CH)])


def _combine_call(y_sorted, s0, s1, w0, w1):
    mesh = plsc.VectorSubcoreMesh(core_axis_name="c", subcore_axis_name="s")
    fn = functools.partial(
        pl.kernel,
        out_type=jax.ShapeDtypeStruct((T, D), jnp.float32),
        mesh=mesh,
        scratch_types=[
            pltpu.VMEM((2, TPW), jnp.int32),
            pltpu.VMEM((TPW,), jnp.float32),
            pltpu.VMEM((TPW,), jnp.float32),
            pltpu.VMEM((CH, D), jnp.float32),
            pltpu.VMEM((CH, D), jnp.float32),
            pltpu.SemaphoreType.DMA,
        ],
        compiler_params=pltpu.CompilerParams(needs_layout_passes=False),
    )(_combine_body)
    return fn(y_sorted, s0, s1, w0, w1)


def kernel(x, router_W, router_b, W1, b1, W2, b2):
    xs = x.reshape(T, D)
    e0, e1, w0, w1, aux = _router_call(xs, router_W, router_b)
    x_sorted, s0, s1, be = _dispatch_call(xs, e0.reshape(T), e1.reshape(T))
    y_sorted = _gffn_call(be, x_sorted, W1, b1, W2, b2)
    out = _combine_call(y_sorted, s0, s1, w0.reshape(T), w1.reshape(T))
    return out.reshape(x.shape), aux[0, 0]


# speedup vs baseline: 3.8778x; 1.1689x over previous
import functools

import jax
import jax.numpy as jnp
from jax import lax
from jax.experimental import pallas as pl
from jax.experimental.pallas import tpu as pltpu
from jax.experimental.pallas import tpu_sc as plsc

T = 2048
D = 1024
F = 4096
E = 8
FB = 512
NF = F // FB

BM = 512
NBLK = (T * 2) // BM + E
NSLOT = NBLK * BM

NC = 2
NS = 16
NW = NC * NS
TPW = T // NW
L = 16


def _gelu_exact(h):
    return h * 0.5 * (1.0 + jax.lax.erf(h * (2.0 ** -0.5)))


def _router_body(x_ref, rw_ref, rb_ref, e0_ref, e1_ref, w0_ref, w1_ref, aux_ref):
    logits = jnp.dot(x_ref[...], rw_ref[...], preferred_element_type=jnp.float32)
    logits = logits + rb_ref[...]
    eio = jax.lax.broadcasted_iota(jnp.int32, (T, E), 1)
    m0 = jnp.max(logits, axis=1, keepdims=True)
    e0 = jnp.min(jnp.where(logits == m0, eio, E), axis=1, keepdims=True)
    masked = jnp.where(eio == e0, -jnp.inf, logits)
    m1 = jnp.max(masked, axis=1, keepdims=True)
    e1 = jnp.min(jnp.where(masked == m1, eio, E), axis=1, keepdims=True)
    t = jnp.exp(m1 - m0)
    w0 = 1.0 / (1.0 + t)
    w1 = 1.0 - w0
    e0_ref[...] = e0
    e1_ref[...] = e1
    w0_ref[...] = w0
    w1_ref[...] = w1
    c0 = jnp.sum((eio == e0).astype(jnp.float32), axis=0, keepdims=True)
    c1 = jnp.sum((eio == e1).astype(jnp.float32), axis=0, keepdims=True)
    mean = (jnp.sum(c0) + jnp.sum(c1)) / (2.0 * E)
    var = (jnp.sum((c0 - mean) ** 2) + jnp.sum((c1 - mean) ** 2)) / (2.0 * E - 1.0)
    aux_ref[...] = jnp.reshape(var / mean * 0.01, (1, 1))


def _router_call(xs, router_W, router_b):
    return pl.pallas_call(
        _router_body,
        grid=(1,),
        in_specs=[
            pl.BlockSpec((T, D), lambda i: (0, 0)),
            pl.BlockSpec((D, E), lambda i: (0, 0)),
            pl.BlockSpec((1, E), lambda i: (0, 0)),
        ],
        out_specs=[
            pl.BlockSpec((T, 1), lambda i: (0, 0)),
            pl.BlockSpec((T, 1), lambda i: (0, 0)),
            pl.BlockSpec((T, 1), lambda i: (0, 0)),
            pl.BlockSpec((T, 1), lambda i: (0, 0)),
            pl.BlockSpec((1, 1), lambda i: (0, 0)),
        ],
        out_shape=[
            jax.ShapeDtypeStruct((T, 1), jnp.int32),
            jax.ShapeDtypeStruct((T, 1), jnp.int32),
            jax.ShapeDtypeStruct((T, 1), jnp.float32),
            jax.ShapeDtypeStruct((T, 1), jnp.float32),
            jax.ShapeDtypeStruct((1, 1), jnp.float32),
        ],
    )(xs, router_W, router_b.reshape(1, E))


def _dispatch_body(x_hbm, e0_hbm, e1_hbm,
                   xs_hbm, s0_hbm, s1_hbm, be_hbm,
                   e0_v, e1_v, slots_v, xrows_v, be_v, sem):
    wid = lax.axis_index("s") * NC + lax.axis_index("c")
    base = wid * TPW
    my_first = wid * (TPW // L)
    lanes = jax.lax.broadcasted_iota(jnp.int32, (L,), 0)
    zeros = jnp.zeros((L,), jnp.int32)

    pltpu.sync_copy(e0_hbm, e0_v)
    pltpu.sync_copy(e1_hbm, e1_v)

    def chunk_hist(v):
        hist = jnp.zeros((L,), jnp.int32)
        for e in range(E):
            cnt = jnp.full((L,), jnp.sum((v == e).astype(jnp.int32)))
            hist = hist + jnp.where(lanes == e, cnt, zeros)
        return hist

    def count_step(c, carry):
        run, mybase = carry
        snap = jnp.full((L,), c) == jnp.full((L,), my_first)
        mybase = jnp.where(snap, run, mybase)
        h = chunk_hist(e0_v[pl.ds(c * L, L)]) + chunk_hist(e1_v[pl.ds(c * L, L)])
        return run + h, mybase

    tot, mybase = lax.fori_loop(0, T // L, count_step, (zeros, zeros))
    capb = (tot + (BM - 1)) // BM
    endb = lax.cumsum(capb)
    offb = endb - capb
    start = offb * BM + mybase

    @pl.when(wid == 0)
    def _():
        for half in range(NBLK // L + 1):
            bi = lanes + half * L
            acc = jnp.zeros((L,), jnp.int32)
            for e in range(E):
                endb_e = jnp.full((L,), jnp.sum(jnp.where(lanes == e, endb, zeros)))
                acc = acc + (bi >= endb_e).astype(jnp.int32)
            be_v[pl.ds(half * L, L)] = jnp.minimum(acc, jnp.full((L,), E - 1))
        pltpu.sync_copy(be_v, be_hbm)

    run2 = start
    for cc in range(TPW // L):
        c = my_first + cc
        for row, src_v in ((0, e0_v), (1, e1_v)):
            v = src_v[pl.ds(c * L, L)]
            r = jnp.zeros((L,), jnp.int32)
            sb = jnp.zeros((L,), jnp.int32)
            hist = jnp.zeros((L,), jnp.int32)
            for e in range(E):
                m = v == e
                cs = lax.cumsum(m.astype(jnp.int32))
                r = jnp.where(m, cs - 1, r)
                run_e = jnp.full((L,), jnp.sum(jnp.where(lanes == e, run2, zeros)))
                sb = jnp.where(m, run_e, sb)
                cnt = jnp.full((L,), jnp.sum(m.astype(jnp.int32)))
                hist = hist + jnp.where(lanes == e, cnt, zeros)
            slots_v[row, pl.ds(cc * L, L)] = sb + r
            run2 = run2 + hist

    pltpu.sync_copy(slots_v.at[0], s0_hbm.at[pl.ds(base, TPW)])
    pltpu.sync_copy(slots_v.at[1], s1_hbm.at[pl.ds(base, TPW)])

    pltpu.sync_copy(x_hbm.at[pl.ds(base, TPW)], xrows_v)
    pltpu.async_copy(xrows_v, xs_hbm.at[slots_v.at[0]], sem).wait()
    pltpu.async_copy(xrows_v, xs_hbm.at[slots_v.at[1]], sem).wait()


def _dispatch_call(xs, e0, e1):
    mesh = plsc.VectorSubcoreMesh(core_axis_name="c", subcore_axis_name="s")
    fn = functools.partial(
        pl.kernel,
        out_type=[
            jax.ShapeDtypeStruct((NSLOT, D), jnp.float32),
            jax.ShapeDtypeStruct((T,), jnp.int32),
            jax.ShapeDtypeStruct((T,), jnp.int32),
            jax.ShapeDtypeStruct((2 * L,), jnp.int32),
        ],
        mesh=mesh,
        scratch_types=[
            pltpu.VMEM((T,), jnp.int32),
            pltpu.VMEM((T,), jnp.int32),
            pltpu.VMEM((2, TPW), jnp.int32),
            pltpu.VMEM((TPW, D), jnp.float32),
            pltpu.VMEM((2 * L,), jnp.int32),
            pltpu.SemaphoreType.DMA,
        ],
        compiler_params=pltpu.CompilerParams(needs_layout_passes=False),
    )(_dispatch_body)
    return fn(xs, e0, e1)


FH = F // 2


def _gffn_body(be_ref, xs_ref, w1_ref, b1_ref, w2_ref, b2_ref, out_ref):
    j = pl.program_id(1)
    h = jnp.dot(xs_ref[...], w1_ref[0], preferred_element_type=jnp.float32)
    h = _gelu_exact(h + b1_ref[0])
    y = jnp.dot(h, w2_ref[0], preferred_element_type=jnp.float32)

    @pl.when(j == 0)
    def _():
        out_ref[...] = y + b2_ref[0]

    @pl.when(j == 1)
    def _():
        out_ref[...] += y


def _gffn_call(be, x_sorted, W1, b1, W2, b2):
    def _jser(blk, j):
        return jax.lax.bitwise_xor(jax.lax.rem(blk, 2), j)

    grid_spec = pltpu.PrefetchScalarGridSpec(
        num_scalar_prefetch=1,
        grid=(NBLK, 2),
        in_specs=[
            pl.BlockSpec((BM, D), lambda blk, j, be: (blk, 0)),
            pl.BlockSpec((1, D, FH), lambda blk, j, be: (be[blk], 0, _jser(blk, j))),
            pl.BlockSpec((1, 1, FH), lambda blk, j, be: (be[blk], 0, _jser(blk, j))),
            pl.BlockSpec((1, FH, D), lambda blk, j, be: (be[blk], _jser(blk, j), 0)),
            pl.BlockSpec((1, 1, D), lambda blk, j, be: (be[blk], 0, 0)),
        ],
        out_specs=pl.BlockSpec((BM, D), lambda blk, j, be: (blk, 0)),
    )
    return pl.pallas_call(
        _gffn_body,
        grid_spec=grid_spec,
        out_shape=jax.ShapeDtypeStruct((NSLOT, D), jnp.float32),
        compiler_params=pltpu.CompilerParams(vmem_limit_bytes=60 * 1024 * 1024),
    )(be, x_sorted, W1, b1.reshape(E, 1, F), W2, b2.reshape(E, 1, D))


CH = 16


def _combine_body(y_hbm, s0_hbm, s1_hbm, w0_hbm, w1_hbm, out_hbm,
                  sl_v, w0_v, w1_v, ya_v, yb_v, sem):
    wid = lax.axis_index("s") * NC + lax.axis_index("c")
    base = wid * TPW
    lanes = jax.lax.broadcasted_iota(jnp.int32, (L,), 0)

    pltpu.sync_copy(s0_hbm.at[pl.ds(base, TPW)], sl_v.at[0])
    pltpu.sync_copy(s1_hbm.at[pl.ds(base, TPW)], sl_v.at[1])
    pltpu.sync_copy(w0_hbm.at[pl.ds(base, TPW)], w0_v)
    pltpu.sync_copy(w1_hbm.at[pl.ds(base, TPW)], w1_v)

    for g in range(TPW // CH):
        ca = pltpu.async_copy(y_hbm.at[sl_v.at[0, pl.ds(g * CH, CH)]], ya_v, sem)
        cb = pltpu.async_copy(y_hbm.at[sl_v.at[1, pl.ds(g * CH, CH)]], yb_v, sem)
        ca.wait()
        cb.wait()
        wa = w0_v[pl.ds(g * CH, CH)]
        wb = w1_v[pl.ds(g * CH, CH)]

        def tok_body(i, _, wa=wa, wb=wb):
            iv = jnp.full((L,), i)
            fz = jnp.zeros((L,), jnp.float32)
            was = jnp.full((L,), jnp.sum(jnp.where(lanes == iv, wa, fz)))
            wbs = jnp.full((L,), jnp.sum(jnp.where(lanes == iv, wb, fz)))
            for j in range(D // L):
                sla = ya_v[i, pl.ds(j * L, L)]
                slb = yb_v[i, pl.ds(j * L, L)]
                ya_v[i, pl.ds(j * L, L)] = was * sla + wbs * slb
            return 0

        lax.fori_loop(0, CH, tok_body, 0)
        pltpu.sync_copy(ya_v, out_hbm.at[pl.ds(base + g * CH, CH)])


def _combine_call(y_sorted, s0, s1, w0, w1):
    mesh = plsc.VectorSubcoreMesh(core_axis_name="c", subcore_axis_name="s")
    fn = functools.partial(
        pl.kernel,
        out_type=jax.ShapeDtypeStruct((T, D), jnp.float32),
        mesh=mesh,
        scratch_types=[
            pltpu.VMEM((2, TPW), jnp.int32),
            pltpu.VMEM((TPW,), jnp.float32),
            pltpu.VMEM((TPW,), jnp.float32),
            pltpu.VMEM((CH, D), jnp.float32),
            pltpu.VMEM((CH, D), jnp.float32),
            pltpu.SemaphoreType.DMA,
        ],
        compiler_params=pltpu.CompilerParams(needs_layout_passes=False),
    )(_combine_body)
    return fn(y_sorted, s0, s1, w0, w1)


def kernel(x, router_W, router_b, W1, b1, W2, b2):
    xs = x.reshape(T, D)
    e0, e1, w0, w1, aux = _router_call(xs, router_W, router_b)
    x_sorted, s0, s1, be = _dispatch_call(xs, e0.reshape(T), e1.reshape(T))
    y_sorted = _gffn_call(be, x_sorted, W1, b1, W2, b2)
    out = _combine_call(y_sorted, s0, s1, w0.reshape(T), w1.reshape(T))
    return out.reshape(x.shape), aux[0, 0]


# speedup vs baseline: 4.1501x; 1.0702x over previous
import functools

import jax
import jax.numpy as jnp
from jax import lax
from jax.experimental import pallas as pl
from jax.experimental.pallas import tpu as pltpu
from jax.experimental.pallas import tpu_sc as plsc

T = 2048
D = 1024
F = 4096
E = 8
FB = 512
NF = F // FB

BM = 512
NBLK = (T * 2) // BM + E
NSLOT = NBLK * BM

NC = 2
NS = 16
NW = NC * NS
TPW = T // NW
L = 16


def _gelu_exact(h):
    return h * 0.5 * (1.0 + jax.lax.erf(h * (2.0 ** -0.5)))


def _router_body(x_ref, rw_ref, rb_ref, e0_ref, e1_ref, w0_ref, w1_ref, aux_ref):
    logits = jnp.dot(x_ref[...], rw_ref[...], preferred_element_type=jnp.float32)
    logits = logits + rb_ref[...]
    eio = jax.lax.broadcasted_iota(jnp.int32, (T, E), 1)
    m0 = jnp.max(logits, axis=1, keepdims=True)
    e0 = jnp.min(jnp.where(logits == m0, eio, E), axis=1, keepdims=True)
    masked = jnp.where(eio == e0, -jnp.inf, logits)
    m1 = jnp.max(masked, axis=1, keepdims=True)
    e1 = jnp.min(jnp.where(masked == m1, eio, E), axis=1, keepdims=True)
    t = jnp.exp(m1 - m0)
    w0 = 1.0 / (1.0 + t)
    w1 = 1.0 - w0
    e0_ref[...] = e0
    e1_ref[...] = e1
    w0_ref[...] = w0
    w1_ref[...] = w1
    c0 = jnp.sum((eio == e0).astype(jnp.float32), axis=0, keepdims=True)
    c1 = jnp.sum((eio == e1).astype(jnp.float32), axis=0, keepdims=True)
    mean = (jnp.sum(c0) + jnp.sum(c1)) / (2.0 * E)
    var = (jnp.sum((c0 - mean) ** 2) + jnp.sum((c1 - mean) ** 2)) / (2.0 * E - 1.0)
    aux_ref[...] = jnp.reshape(var / mean * 0.01, (1, 1))


def _router_call(xs, router_W, router_b):
    return pl.pallas_call(
        _router_body,
        grid=(1,),
        in_specs=[
            pl.BlockSpec((T, D), lambda i: (0, 0)),
            pl.BlockSpec((D, E), lambda i: (0, 0)),
            pl.BlockSpec((1, E), lambda i: (0, 0)),
        ],
        out_specs=[
            pl.BlockSpec((T, 1), lambda i: (0, 0)),
            pl.BlockSpec((T, 1), lambda i: (0, 0)),
            pl.BlockSpec((T, 1), lambda i: (0, 0)),
            pl.BlockSpec((T, 1), lambda i: (0, 0)),
            pl.BlockSpec((1, 1), lambda i: (0, 0)),
        ],
        out_shape=[
            jax.ShapeDtypeStruct((T, 1), jnp.int32),
            jax.ShapeDtypeStruct((T, 1), jnp.int32),
            jax.ShapeDtypeStruct((T, 1), jnp.float32),
            jax.ShapeDtypeStruct((T, 1), jnp.float32),
            jax.ShapeDtypeStruct((1, 1), jnp.float32),
        ],
    )(xs, router_W, router_b.reshape(1, E))


def _dispatch_body(x_hbm, e0_hbm, e1_hbm,
                   xs_hbm, s0_hbm, s1_hbm, be_hbm,
                   e0_v, e1_v, slots_v, xrows_v, be_v, sem):
    wid = lax.axis_index("s") * NC + lax.axis_index("c")
    base = wid * TPW
    my_first = wid * (TPW // L)
    lanes = jax.lax.broadcasted_iota(jnp.int32, (L,), 0)
    zeros = jnp.zeros((L,), jnp.int32)

    pltpu.sync_copy(e0_hbm, e0_v)
    pltpu.sync_copy(e1_hbm, e1_v)

    def chunk_hist(v):
        hist = jnp.zeros((L,), jnp.int32)
        for e in range(E):
            cnt = jnp.full((L,), jnp.sum((v == e).astype(jnp.int32)))
            hist = hist + jnp.where(lanes == e, cnt, zeros)
        return hist

    def count_step(c, carry):
        run, mybase = carry
        snap = jnp.full((L,), c) == jnp.full((L,), my_first)
        mybase = jnp.where(snap, run, mybase)
        h = chunk_hist(e0_v[pl.ds(c * L, L)]) + chunk_hist(e1_v[pl.ds(c * L, L)])
        return run + h, mybase

    tot, mybase = lax.fori_loop(0, T // L, count_step, (zeros, zeros))
    capb = (tot + (BM - 1)) // BM
    endb = lax.cumsum(capb)
    offb = endb - capb
    start = offb * BM + mybase

    @pl.when(wid == 0)
    def _():
        for half in range(NBLK // L + 1):
            bi = lanes + half * L
            acc = jnp.zeros((L,), jnp.int32)
            for e in range(E):
                endb_e = jnp.full((L,), jnp.sum(jnp.where(lanes == e, endb, zeros)))
                acc = acc + (bi >= endb_e).astype(jnp.int32)
            be_v[pl.ds(half * L, L)] = acc
        pltpu.sync_copy(be_v, be_hbm)

    run2 = start
    for cc in range(TPW // L):
        c = my_first + cc
        for row, src_v in ((0, e0_v), (1, e1_v)):
            v = src_v[pl.ds(c * L, L)]
            r = jnp.zeros((L,), jnp.int32)
            sb = jnp.zeros((L,), jnp.int32)
            hist = jnp.zeros((L,), jnp.int32)
            for e in range(E):
                m = v == e
                cs = lax.cumsum(m.astype(jnp.int32))
                r = jnp.where(m, cs - 1, r)
                run_e = jnp.full((L,), jnp.sum(jnp.where(lanes == e, run2, zeros)))
                sb = jnp.where(m, run_e, sb)
                cnt = jnp.full((L,), jnp.sum(m.astype(jnp.int32)))
                hist = hist + jnp.where(lanes == e, cnt, zeros)
            slots_v[row, pl.ds(cc * L, L)] = sb + r
            run2 = run2 + hist

    pltpu.sync_copy(slots_v.at[0], s0_hbm.at[pl.ds(base, TPW)])
    pltpu.sync_copy(slots_v.at[1], s1_hbm.at[pl.ds(base, TPW)])

    pltpu.sync_copy(x_hbm.at[pl.ds(base, TPW)], xrows_v)
    pltpu.async_copy(xrows_v, xs_hbm.at[slots_v.at[0]], sem).wait()
    pltpu.async_copy(xrows_v, xs_hbm.at[slots_v.at[1]], sem).wait()


def _dispatch_call(xs, e0, e1):
    mesh = plsc.VectorSubcoreMesh(core_axis_name="c", subcore_axis_name="s")
    fn = functools.partial(
        pl.kernel,
        out_type=[
            jax.ShapeDtypeStruct((NSLOT, D), jnp.float32),
            jax.ShapeDtypeStruct((T,), jnp.int32),
            jax.ShapeDtypeStruct((T,), jnp.int32),
            jax.ShapeDtypeStruct((2 * L,), jnp.int32),
        ],
        mesh=mesh,
        scratch_types=[
            pltpu.VMEM((T,), jnp.int32),
            pltpu.VMEM((T,), jnp.int32),
            pltpu.VMEM((2, TPW), jnp.int32),
            pltpu.VMEM((TPW, D), jnp.float32),
            pltpu.VMEM((2 * L,), jnp.int32),
            pltpu.SemaphoreType.DMA,
        ],
        compiler_params=pltpu.CompilerParams(needs_layout_passes=False),
    )(_dispatch_body)
    return fn(xs, e0, e1)


FH = F // 2


def _gffn_body(be_ref, xs_ref, w1_ref, b1_ref, w2_ref, b2_ref, out_ref):
    blk = pl.program_id(0)
    j = pl.program_id(1)

    @pl.when(be_ref[blk] < E)
    def _():
        h = jnp.dot(xs_ref[...], w1_ref[0], preferred_element_type=jnp.float32)
        h = _gelu_exact(h + b1_ref[0])
        y = jnp.dot(h, w2_ref[0], preferred_element_type=jnp.float32)

        @pl.when(j == 0)
        def _():
            out_ref[...] = y + b2_ref[0]

        @pl.when(j == 1)
        def _():
            out_ref[...] += y


def _gffn_call(be, x_sorted, W1, b1, W2, b2):
    def _jser(blk, j):
        return jax.lax.bitwise_xor(jax.lax.rem(blk, 2), j)

    grid_spec = pltpu.PrefetchScalarGridSpec(
        num_scalar_prefetch=1,
        grid=(NBLK, 2),
        in_specs=[
            pl.BlockSpec((BM, D), lambda blk, j, be: (blk, 0)),
            pl.BlockSpec((1, D, FH),
                         lambda blk, j, be: (jnp.minimum(be[blk], E - 1), 0,
                                             _jser(blk, j))),
            pl.BlockSpec((1, 1, FH),
                         lambda blk, j, be: (jnp.minimum(be[blk], E - 1), 0,
                                             _jser(blk, j))),
            pl.BlockSpec((1, FH, D),
                         lambda blk, j, be: (jnp.minimum(be[blk], E - 1),
                                             _jser(blk, j), 0)),
            pl.BlockSpec((1, 1, D),
                         lambda blk, j, be: (jnp.minimum(be[blk], E - 1), 0, 0)),
        ],
        out_specs=pl.BlockSpec((BM, D), lambda blk, j, be: (blk, 0)),
    )
    return pl.pallas_call(
        _gffn_body,
        grid_spec=grid_spec,
        out_shape=jax.ShapeDtypeStruct((NSLOT, D), jnp.float32),
        compiler_params=pltpu.CompilerParams(vmem_limit_bytes=60 * 1024 * 1024),
    )(be, x_sorted, W1, b1.reshape(E, 1, F), W2, b2.reshape(E, 1, D))


CH = 16


def _combine_body(y_hbm, s0_hbm, s1_hbm, w0_hbm, w1_hbm, out_hbm,
                  sl_v, w0_v, w1_v, ya_v, yb_v, sem):
    wid = lax.axis_index("s") * NC + lax.axis_index("c")
    base = wid * TPW
    lanes = jax.lax.broadcasted_iota(jnp.int32, (L,), 0)

    pltpu.sync_copy(s0_hbm.at[pl.ds(base, TPW)], sl_v.at[0])
    pltpu.sync_copy(s1_hbm.at[pl.ds(base, TPW)], sl_v.at[1])
    pltpu.sync_copy(w0_hbm.at[pl.ds(base, TPW)], w0_v)
    pltpu.sync_copy(w1_hbm.at[pl.ds(base, TPW)], w1_v)

    for g in range(TPW // CH):
        ca = pltpu.async_copy(y_hbm.at[sl_v.at[0, pl.ds(g * CH, CH)]], ya_v, sem)
        cb = pltpu.async_copy(y_hbm.at[sl_v.at[1, pl.ds(g * CH, CH)]], yb_v, sem)
        ca.wait()
        cb.wait()
        wa = w0_v[pl.ds(g * CH, CH)]
        wb = w1_v[pl.ds(g * CH, CH)]

        def tok_body(i, _, wa=wa, wb=wb):
            iv = jnp.full((L,), i)
            fz = jnp.zeros((L,), jnp.float32)
            was = jnp.full((L,), jnp.sum(jnp.where(lanes == iv, wa, fz)))
            wbs = jnp.full((L,), jnp.sum(jnp.where(lanes == iv, wb, fz)))
            for j in range(D // L):
                sla = ya_v[i, pl.ds(j * L, L)]
                slb = yb_v[i, pl.ds(j * L, L)]
                ya_v[i, pl.ds(j * L, L)] = was * sla + wbs * slb
            return 0

        lax.fori_loop(0, CH, tok_body, 0)
        pltpu.sync_copy(ya_v, out_hbm.at[pl.ds(base + g * CH, CH)])


def _combine_call(y_sorted, s0, s1, w0, w1):
    mesh = plsc.VectorSubcoreMesh(core_axis_name="c", subcore_axis_name="s")
    fn = functools.partial(
        pl.kernel,
        out_type=jax.ShapeDtypeStruct((T, D), jnp.float32),
        mesh=mesh,
        scratch_types=[
            pltpu.VMEM((2, TPW), jnp.int32),
            pltpu.VMEM((TPW,), jnp.float32),
            pltpu.VMEM((TPW,), jnp.float32),
            pltpu.VMEM((CH, D), jnp.float32),
            pltpu.VMEM((CH, D), jnp.float32),
            pltpu.SemaphoreType.DMA,
        ],
        compiler_params=pltpu.CompilerParams(needs_layout_passes=False),
    )(_combine_body)
    return fn(y_sorted, s0, s1, w0, w1)


def kernel(x, router_W, router_b, W1, b1, W2, b2):
    xs = x.reshape(T, D)
    e0, e1, w0, w1, aux = _router_call(xs, router_W, router_b)
    x_sorted, s0, s1, be = _dispatch_call(xs, e0.reshape(T), e1.reshape(T))
    y_sorted = _gffn_call(be, x_sorted, W1, b1, W2, b2)
    out = _combine_call(y_sorted, s0, s1, w0.reshape(T), w1.reshape(T))
    return out.reshape(x.shape), aux[0, 0]
